# baseline (device time: 202136 ns/iter reference)
import jax
import jax.numpy as jnp
from jax import lax
from jax.experimental import pallas as pl
from jax.experimental.pallas import tpu as pltpu

N_DEV = 4
N_TILE = 2


def kernel(O, Wo):
    B, S, Hs, D = O.shape
    K = Hs * D
    F = Wo.shape[1]
    s_chunk = S // N_DEV
    s_tile = s_chunk // N_TILE

    Wo16 = Wo.astype(jnp.bfloat16)

    def body(o_hbm, wo_ref, out_ref, comm_ref, o_stage, pc_ref,
             send_sems, recv_sems, hop0_send_sems, hop0_recv_sems,
             o_sems, out_sem):
        me = lax.axis_index("i")
        left = (me - 1) % N_DEV
        right = (me + 1) % N_DEV

        barrier_sem = pltpu.get_barrier_semaphore()
        for nbr in [left, right]:
            pl.semaphore_signal(
                barrier_sem, inc=1,
                device_id=(nbr,), device_id_type=pl.DeviceIdType.MESH,
            )
        pl.semaphore_wait(barrier_sem, 2)

        def chunk_at(d, i):
            return (me - 1 - i) % N_DEV if d == 0 else (me + 1 + i) % N_DEV

        def start_o_load(d, i):
            cp = pltpu.make_async_copy(
                o_hbm.at[d, pl.ds(chunk_at(d, i) * s_chunk, s_chunk)],
                o_stage.at[d],
                o_sems.at[d],
            )
            cp.start()
            return cp

        def compute_tile(d, t, dst):
            o_tile = (
                o_stage[d, t * s_tile:(t + 1) * s_tile]
                .astype(jnp.bfloat16)
                .reshape(s_tile, K)
            )
            dst[pl.ds(t * s_tile, s_tile), :] = jnp.dot(
                o_tile, wo_ref[...], preferred_element_type=jnp.float32
            ).astype(jnp.bfloat16)

        def make_rdma(d, h):
            return pltpu.make_async_remote_copy(
                src_ref=comm_ref.at[h, d],
                dst_ref=comm_ref.at[h + 1, d],
                send_sem=send_sems.at[h, d],
                recv_sem=recv_sems.at[h + 1, d],
                device_id=(right if d == 0 else left,),
                device_id_type=pl.DeviceIdType.MESH,
            )

        def make_tile_rdma(d, t):
            rows = pl.ds(t * s_tile, s_tile)
            return pltpu.make_async_remote_copy(
                src_ref=comm_ref.at[0, d, rows],
                dst_ref=comm_ref.at[1, d, rows],
                send_sem=hop0_send_sems.at[d, t],
                recv_sem=hop0_recv_sems.at[d, t],
                device_id=(right if d == 0 else left,),
                device_id_type=pl.DeviceIdType.MESH,
            )

        loads = {(0, 0): start_o_load(0, 0), (1, 0): start_o_load(1, 0)}

        all_rdmas = []
        hop0_rdmas = []
        for t in range(N_TILE):
            for d in range(2):
                if t == 0:
                    loads[(d, 0)].wait()
                compute_tile(d, t, comm_ref.at[0, d])
                r = make_tile_rdma(d, t)
                r.start()
                hop0_rdmas.append(r)
                all_rdmas.append(r)
        for d in range(2):
            loads[(d, 1)] = start_o_load(d, 1)

        hop_rdmas = {}
        for h in range(N_DEV - 1):
            for d in range(2):
                loads[(d, h + 1)].wait()
                for t in range(N_TILE):
                    compute_tile(d, t, pc_ref.at[d])
                if h < N_DEV - 2:
                    loads[(d, h + 2)] = start_o_load(d, h + 2)
            next_rdmas = {}
            for d in range(2):
                if h == 0:
                    hop0_rdmas[d].wait_recv()
                    hop0_rdmas[2 + d].wait_recv()
                else:
                    hop_rdmas[d].wait_recv()
                comm_ref[h + 1, d] = comm_ref[h + 1, d] + pc_ref[d]
                if h < N_DEV - 2:
                    r = make_rdma(d, h + 1)
                    r.start()
                    next_rdmas[d] = r
                    all_rdmas.append(r)
            hop_rdmas = next_rdmas

        out_cps = [
            pltpu.make_async_copy(
                comm_ref.at[N_DEV - 1, d], out_ref.at[d], out_sem
            )
            for d in range(2)
        ]
        for cp in out_cps:
            cp.start()

        for r in all_rdmas:
            r.wait_send()
        for cp in out_cps:
            cp.wait()

    return pl.pallas_call(
        body,
        out_shape=jax.ShapeDtypeStruct((B, s_chunk, F), jnp.bfloat16),
        in_specs=[
            pl.BlockSpec(memory_space=pl.ANY),
            pl.BlockSpec(memory_space=pltpu.VMEM),
        ],
        out_specs=pl.BlockSpec(memory_space=pl.ANY),
        scratch_shapes=[
            pltpu.VMEM((N_DEV, B, s_chunk, F), jnp.bfloat16),
            pltpu.VMEM((2, s_chunk, Hs, D), jnp.float32),
            pltpu.VMEM((2, s_chunk, F), jnp.bfloat16),
            pltpu.SemaphoreType.DMA((N_DEV, 2)),
            pltpu.SemaphoreType.DMA((N_DEV, 2)),
            pltpu.SemaphoreType.DMA((2, N_TILE)),
            pltpu.SemaphoreType.DMA((2, N_TILE)),
            pltpu.SemaphoreType.DMA((2,)),
            pltpu.SemaphoreType.DMA,
        ],
        compiler_params=pltpu.CompilerParams(
            collective_id=0, vmem_limit_bytes=100 * 1024 * 1024
        ),
    )(O, Wo16)


# device time: 186400 ns/iter; 1.0844x vs baseline; 1.0844x over previous
import jax
import jax.numpy as jnp
from jax import lax
from jax.experimental import pallas as pl
from jax.experimental.pallas import tpu as pltpu

N_DEV = 4
N_TILE = 2


def kernel(O, Wo):
    B, S, Hs, D = O.shape
    K = Hs * D
    F = Wo.shape[1]
    s_chunk = S // N_DEV
    s_tile = s_chunk // N_TILE

    O3 = O.astype(jnp.bfloat16).reshape(B, S, K)
    Wo16 = Wo.astype(jnp.bfloat16)

    def body(o_hbm, wo_ref, out_ref, comm_ref, o_stage, pc_ref,
             send_sems, recv_sems, hop0_send_sems, hop0_recv_sems,
             o_sems, out_sem):
        me = lax.axis_index("i")
        left = (me - 1) % N_DEV
        right = (me + 1) % N_DEV

        barrier_sem = pltpu.get_barrier_semaphore()
        for nbr in [left, right]:
            pl.semaphore_signal(
                barrier_sem, inc=1,
                device_id=(nbr,), device_id_type=pl.DeviceIdType.MESH,
            )
        pl.semaphore_wait(barrier_sem, 2)

        def chunk_at(d, i):
            return (me - 1 - i) % N_DEV if d == 0 else (me + 1 + i) % N_DEV

        def start_o_load(d, i):
            buf = d * 2 + i % 2
            cp = pltpu.make_async_copy(
                o_hbm.at[d, pl.ds(chunk_at(d, i) * s_chunk, s_chunk), :],
                o_stage.at[buf],
                o_sems.at[buf],
            )
            cp.start()
            return cp

        def compute_pc(d, i, dst):
            buf = d * 2 + i % 2
            for t in range(N_TILE):
                rows = pl.ds(t * s_tile, s_tile)
                dst[rows, :] = jnp.dot(
                    o_stage[buf, t * s_tile:(t + 1) * s_tile, :],
                    wo_ref[...],
                    preferred_element_type=jnp.float32,
                ).astype(jnp.bfloat16)

        def make_rdma(d, h):
            return pltpu.make_async_remote_copy(
                src_ref=comm_ref.at[h, d],
                dst_ref=comm_ref.at[h + 1, d],
                send_sem=send_sems.at[h, d],
                recv_sem=recv_sems.at[h + 1, d],
                device_id=(right if d == 0 else left,),
                device_id_type=pl.DeviceIdType.MESH,
            )

        def compute_tile(d, i, t, dst):
            buf = d * 2 + i % 2
            dst[pl.ds(t * s_tile, s_tile), :] = jnp.dot(
                o_stage[buf, t * s_tile:(t + 1) * s_tile, :],
                wo_ref[...],
                preferred_element_type=jnp.float32,
            ).astype(jnp.bfloat16)

        def make_tile_rdma(d, t):
            rows = pl.ds(t * s_tile, s_tile)
            return pltpu.make_async_remote_copy(
                src_ref=comm_ref.at[0, d, rows],
                dst_ref=comm_ref.at[1, d, rows],
                send_sem=hop0_send_sems.at[d, t],
                recv_sem=hop0_recv_sems.at[d, t],
                device_id=(right if d == 0 else left,),
                device_id_type=pl.DeviceIdType.MESH,
            )

        loads = {(0, 0): start_o_load(0, 0), (1, 0): start_o_load(1, 0)}

        all_rdmas = []
        hop0_rdmas = []
        for t in range(N_TILE):
            for d in range(2):
                if t == 0:
                    loads[(d, 0)].wait()
                    loads[(d, 1)] = start_o_load(d, 1)
                compute_tile(d, 0, t, comm_ref.at[0, d])
                r = make_tile_rdma(d, t)
                r.start()
                hop0_rdmas.append(r)
                all_rdmas.append(r)

        hop_rdmas = {}
        for h in range(N_DEV - 1):
            for d in range(2):
                loads[(d, h + 1)].wait()
                compute_pc(d, h + 1, pc_ref.at[d])
                if h < N_DEV - 2:
                    loads[(d, h + 2)] = start_o_load(d, h + 2)
            next_rdmas = {}
            for d in range(2):
                if h == 0:
                    hop0_rdmas[d].wait_recv()
                    hop0_rdmas[2 + d].wait_recv()
                else:
                    hop_rdmas[d].wait_recv()
                comm_ref[h + 1, d] = comm_ref[h + 1, d] + pc_ref[d]
                if h < N_DEV - 2:
                    r = make_rdma(d, h + 1)
                    r.start()
                    next_rdmas[d] = r
                    all_rdmas.append(r)
            hop_rdmas = next_rdmas

        out_cps = [
            pltpu.make_async_copy(
                comm_ref.at[N_DEV - 1, d], out_ref.at[d], out_sem
            )
            for d in range(2)
        ]
        for cp in out_cps:
            cp.start()

        for r in all_rdmas:
            r.wait_send()
        for cp in out_cps:
            cp.wait()

    return pl.pallas_call(
        body,
        out_shape=jax.ShapeDtypeStruct((B, s_chunk, F), jnp.bfloat16),
        in_specs=[
            pl.BlockSpec(memory_space=pl.ANY),
            pl.BlockSpec(memory_space=pltpu.VMEM),
        ],
        out_specs=pl.BlockSpec(memory_space=pl.ANY),
        scratch_shapes=[
            pltpu.VMEM((N_DEV, B, s_chunk, F), jnp.bfloat16),
            pltpu.VMEM((4, s_chunk, K), jnp.bfloat16),
            pltpu.VMEM((2, s_chunk, F), jnp.bfloat16),
            pltpu.SemaphoreType.DMA((N_DEV, 2)),
            pltpu.SemaphoreType.DMA((N_DEV, 2)),
            pltpu.SemaphoreType.DMA((2, N_TILE)),
            pltpu.SemaphoreType.DMA((2, N_TILE)),
            pltpu.SemaphoreType.DMA((4,)),
            pltpu.SemaphoreType.DMA,
        ],
        compiler_params=pltpu.CompilerParams(
            collective_id=0, vmem_limit_bytes=100 * 1024 * 1024
        ),
    )(O3, Wo16)


# device time: 183541 ns/iter; 1.1013x vs baseline; 1.0156x over previous
import jax
import jax.numpy as jnp
from jax import lax
from jax.experimental import pallas as pl
from jax.experimental.pallas import tpu as pltpu

N_DEV = 4
N_TILE = 2


def kernel(O, Wo):
    B, S, Hs, D = O.shape
    K = Hs * D
    F = Wo.shape[1]
    s_chunk = S // N_DEV
    s_tile = s_chunk // N_TILE

    O3 = O.astype(jnp.bfloat16).reshape(B, S, K)

    def body(o_hbm, wo_hbm, out_ref, comm_ref, wo_ref, o_stage, pc_ref,
             wo_stage, send_sems, recv_sems, hop0_send_sems, hop0_recv_sems,
             o_sems, wo_sems, out_sem):
        me = lax.axis_index("i")
        left = (me - 1) % N_DEV
        right = (me + 1) % N_DEV

        barrier_sem = pltpu.get_barrier_semaphore()
        for nbr in [left, right]:
            pl.semaphore_signal(
                barrier_sem, inc=1,
                device_id=(nbr,), device_id_type=pl.DeviceIdType.MESH,
            )
        pl.semaphore_wait(barrier_sem, 2)

        def chunk_at(d, i):
            return (me - 1 - i) % N_DEV if d == 0 else (me + 1 + i) % N_DEV

        def start_o_load(d, i):
            buf = d * 2 + i % 2
            cp = pltpu.make_async_copy(
                o_hbm.at[d, pl.ds(chunk_at(d, i) * s_chunk, s_chunk), :],
                o_stage.at[buf],
                o_sems.at[buf],
            )
            cp.start()
            return cp

        def compute_pc(d, i, dst):
            buf = d * 2 + i % 2
            for t in range(N_TILE):
                rows = pl.ds(t * s_tile, s_tile)
                dst[rows, :] = jnp.dot(
                    o_stage[buf, t * s_tile:(t + 1) * s_tile, :],
                    wo_ref[...],
                    preferred_element_type=jnp.float32,
                ).astype(jnp.bfloat16)

        def make_rdma(d, h):
            return pltpu.make_async_remote_copy(
                src_ref=comm_ref.at[h, d],
                dst_ref=comm_ref.at[h + 1, d],
                send_sem=send_sems.at[h, d],
                recv_sem=recv_sems.at[h + 1, d],
                device_id=(right if d == 0 else left,),
                device_id_type=pl.DeviceIdType.MESH,
            )

        def compute_tile(d, i, t, dst):
            buf = d * 2 + i % 2
            dst[pl.ds(t * s_tile, s_tile), :] = jnp.dot(
                o_stage[buf, t * s_tile:(t + 1) * s_tile, :],
                wo_ref[...],
                preferred_element_type=jnp.float32,
            ).astype(jnp.bfloat16)

        def make_tile_rdma(d, t):
            rows = pl.ds(t * s_tile, s_tile)
            return pltpu.make_async_remote_copy(
                src_ref=comm_ref.at[0, d, rows],
                dst_ref=comm_ref.at[1, d, rows],
                send_sem=hop0_send_sems.at[d, t],
                recv_sem=hop0_recv_sems.at[d, t],
                device_id=(right if d == 0 else left,),
                device_id_type=pl.DeviceIdType.MESH,
            )

        loads = {(0, 0): start_o_load(0, 0), (1, 0): start_o_load(1, 0)}

        k_slab = K // 8

        def start_wo_load(s):
            cp = pltpu.make_async_copy(
                wo_hbm.at[pl.ds(s * k_slab, k_slab), :],
                wo_stage.at[s % 2],
                wo_sems.at[s % 2],
            )
            cp.start()
            return cp

        wo_loads = {0: start_wo_load(0), 1: start_wo_load(1)}
        for s in range(8):
            wo_loads[s].wait()
            if s + 2 < 8:
                wo_loads[s + 2] = start_wo_load(s + 2)
            wo_ref[pl.ds(s * k_slab, k_slab), :] = wo_stage[s % 2].astype(
                jnp.bfloat16
            )

        all_rdmas = []
        hop0_rdmas = []
        for t in range(N_TILE):
            for d in range(2):
                if t == 0:
                    loads[(d, 0)].wait()
                    loads[(d, 1)] = start_o_load(d, 1)
                compute_tile(d, 0, t, comm_ref.at[0, d])
                r = make_tile_rdma(d, t)
                r.start()
                hop0_rdmas.append(r)
                all_rdmas.append(r)

        hop_rdmas = {}
        for h in range(N_DEV - 1):
            for d in range(2):
                loads[(d, h + 1)].wait()
                compute_pc(d, h + 1, pc_ref.at[d])
                if h < N_DEV - 2:
                    loads[(d, h + 2)] = start_o_load(d, h + 2)
            next_rdmas = {}
            for d in range(2):
                if h == 0:
                    hop0_rdmas[d].wait_recv()
                    hop0_rdmas[2 + d].wait_recv()
                else:
                    hop_rdmas[d].wait_recv()
                comm_ref[h + 1, d] = comm_ref[h + 1, d] + pc_ref[d]
                if h < N_DEV - 2:
                    r = make_rdma(d, h + 1)
                    r.start()
                    next_rdmas[d] = r
                    all_rdmas.append(r)
            hop_rdmas = next_rdmas

        out_cps = [
            pltpu.make_async_copy(
                comm_ref.at[N_DEV - 1, d], out_ref.at[d], out_sem
            )
            for d in range(2)
        ]
        for cp in out_cps:
            cp.start()

        for r in all_rdmas:
            r.wait_send()
        for cp in out_cps:
            cp.wait()

    return pl.pallas_call(
        body,
        out_shape=jax.ShapeDtypeStruct((B, s_chunk, F), jnp.bfloat16),
        in_specs=[
            pl.BlockSpec(memory_space=pl.ANY),
            pl.BlockSpec(memory_space=pl.ANY),
        ],
        out_specs=pl.BlockSpec(memory_space=pl.ANY),
        scratch_shapes=[
            pltpu.VMEM((N_DEV, B, s_chunk, F), jnp.bfloat16),
            pltpu.VMEM((K, F), jnp.bfloat16),
            pltpu.VMEM((4, s_chunk, K), jnp.bfloat16),
            pltpu.VMEM((2, s_chunk, F), jnp.bfloat16),
            pltpu.VMEM((2, K // 8, F), jnp.float32),
            pltpu.SemaphoreType.DMA((N_DEV, 2)),
            pltpu.SemaphoreType.DMA((N_DEV, 2)),
            pltpu.SemaphoreType.DMA((2, N_TILE)),
            pltpu.SemaphoreType.DMA((2, N_TILE)),
            pltpu.SemaphoreType.DMA((4,)),
            pltpu.SemaphoreType.DMA((2,)),
            pltpu.SemaphoreType.DMA,
        ],
        compiler_params=pltpu.CompilerParams(
            collective_id=0, vmem_limit_bytes=100 * 1024 * 1024
        ),
    )(O3, Wo)
